# baseline (speedup 1.0000x reference)
import functools
import math

import jax
import jax.numpy as jnp
from jax import lax
from jax.experimental import pallas as pl
from jax.experimental.pallas import tpu as pltpu
from jax.experimental.pallas import tpu_sc as plsc

VOCAB = 1000000
DIM = 32
ROWS = 4096
COLS = 200
SCALE = math.sqrt(float(DIM))

NW = 32
RB = ROWS // NW
CB = COLS // 8
NBUF = 4
NGRP = COLS // NBUF

_mesh = plsc.VectorSubcoreMesh(core_axis_name="c", subcore_axis_name="s")


def _transpose_scale(src, dst):
    lanes = lax.iota(jnp.int32, 16)

    def body(g, carry):
        for u in range(4):
            r = g * 4 + u
            rvec = jnp.full((16,), r, jnp.int32)
            for h in range(2):
                d = h * 16 + lanes
                ti_idx = lax.shift_right_logical(d, 3)
                f_idx = lax.bitwise_and(d, 7)
                v = src[r, pl.ds(16 * h, 16)]
                plsc.store_scatter(dst, [ti_idx, f_idx, rvec], v)
        return carry

    lax.fori_loop(0, RB // 4, body, 0)


@functools.partial(
    pl.kernel,
    out_type=jax.ShapeDtypeStruct((COLS, 4, NW, 8, RB), jnp.float32),
    mesh=_mesh,
    compiler_params=pltpu.CompilerParams(use_tc_tiling_on_sc=False,
                                         needs_layout_passes=False),
    scratch_types=[
        pltpu.VMEM((CB, 8, RB), jnp.int32),
        *[pltpu.VMEM((RB, DIM), jnp.float32) for _ in range(NBUF)],
        *[pltpu.VMEM((4, 8, RB + 1), jnp.float32) for _ in range(NBUF)],
        *[pltpu.SemaphoreType.DMA for _ in range(NBUF)],
        *[pltpu.SemaphoreType.DMA for _ in range(NBUF)],
    ],
)
def _emb_lookup(x_hbm, table_hbm, out_hbm, idx_v, *bufs_and_sems):
    in_bufs = bufs_and_sems[:NBUF]
    out_bufs = bufs_and_sems[NBUF:2 * NBUF]
    gsems = bufs_and_sems[2 * NBUF:3 * NBUF]
    ssems = bufs_and_sems[3 * NBUF:4 * NBUF]

    wid = lax.axis_index("s") * 2 + lax.axis_index("c")

    pltpu.sync_copy(x_hbm.at[:, wid], idx_v)

    def _remap_idx(r, carry):
        cb = r // 8
        a = r % 8
        for h in range(RB // 16):
            v = idx_v[cb, a, pl.ds(16 * h, 16)]
            q = ((v >= VQ).astype(jnp.int32)
                 + (v >= 2 * VQ).astype(jnp.int32)
                 + (v >= 3 * VQ).astype(jnp.int32))
            idx_v[cb, a, pl.ds(16 * h, 16)] = (v - q * VQ) * 4 + q
        return carry

    lax.fori_loop(0, CB * 8, _remap_idx, 0)

    def _idx_slice(c):
        return idx_v.at[c // 8, c % 8]

    def _out_slice(c):
        return out_hbm.at[c, :, wid]

    for b in range(NBUF):
        pltpu.async_copy(table_hbm.at[_idx_slice(b)],
                         in_bufs[b], gsems[b])

    def group(g, carry):
        for b in range(NBUF):
            c = g * NBUF + b
            pltpu.make_async_copy(
                table_hbm.at[_idx_slice(c)],
                in_bufs[b], gsems[b]).wait()
            @pl.when(g > 0)
            def _():
                pltpu.make_async_copy(out_bufs[b].at[:, :, pl.ds(0, RB)],
                                      _out_slice(c - NBUF), ssems[b]).wait()

            _transpose_scale(in_bufs[b], out_bufs[b])

            @pl.when(g < NGRP - 1)
            def _():
                pltpu.async_copy(
                    table_hbm.at[_idx_slice(c + NBUF)],
                    in_bufs[b], gsems[b])

            pltpu.async_copy(out_bufs[b].at[:, :, pl.ds(0, RB)],
                             _out_slice(c), ssems[b])
        return carry

    lax.fori_loop(0, NGRP, group, 0)

    for b in range(NBUF):
        c = (NGRP - 1) * NBUF + b
        pltpu.make_async_copy(out_bufs[b].at[:, :, pl.ds(0, RB)],
                              _out_slice(c), ssems[b]).wait()


TW = 1024
TGRID = 245
VQ = TW * TGRID


def _tc_transpose_body(t0, t1, t2, t3, out_ref):
    eye = jnp.eye(DIM, dtype=jnp.float32) * SCALE
    dn = (((0,), (0,)), ((), ()))

    def tr(t):
        return lax.dot_general(t[...], eye, dn,
                               precision=lax.Precision.HIGHEST,
                               preferred_element_type=jnp.float32)

    out_ref[...] = jnp.concatenate([tr(t0), tr(t1), tr(t2), tr(t3)], axis=1)


_tc_transpose = pl.pallas_call(
    _tc_transpose_body,
    grid=(TGRID,),
    in_specs=[
        pl.BlockSpec((DIM, TW), functools.partial(
            lambda j, i: (0, jnp.minimum(j * TGRID + i,
                                         (VOCAB - 1) // TW)), j))
        for j in range(4)
    ],
    out_specs=pl.BlockSpec((TW, 4 * DIM), lambda i: (i, 0)),
    out_shape=jax.ShapeDtypeStruct((VQ, 4 * DIM), jnp.float32),
)


def kernel(x, table):
    xp = x.astype(jnp.int32).T.reshape(CB, 8, NW, RB).transpose(0, 2, 1, 3)
    tt = table.T
    t2 = _tc_transpose(tt, tt, tt, tt)
    out5 = _emb_lookup(xp, t2.reshape(4 * VQ, DIM))
    return out5.transpose(2, 4, 0, 1, 3).reshape(ROWS, COLS, DIM)


# speedup vs baseline: 1.5324x; 1.5324x over previous
import functools
import math

import jax
import jax.numpy as jnp
from jax import lax
from jax.experimental import pallas as pl
from jax.experimental.pallas import tpu as pltpu
from jax.experimental.pallas import tpu_sc as plsc

VOCAB = 1000000
DIM = 32
ROWS = 4096
COLS = 200
SCALE = math.sqrt(float(DIM))

NW = 32
RB = ROWS // NW
CB = COLS // 8
NBUF = 4
NGRP = COLS // NBUF

_mesh = plsc.VectorSubcoreMesh(core_axis_name="c", subcore_axis_name="s")


def _transpose_scale(src, dst):
    lanes = lax.iota(jnp.int32, 16)

    def body(g, carry):
        for u in range(4):
            r = g * 4 + u
            rvec = jnp.full((16,), r, jnp.int32)
            for h in range(2):
                d = h * 16 + lanes
                ti_idx = lax.shift_right_logical(d, 3)
                f_idx = lax.bitwise_and(d, 7)
                v = src[r, pl.ds(16 * h, 16)]
                plsc.store_scatter(dst, [ti_idx, f_idx, rvec], v)
        return carry

    lax.fori_loop(0, RB // 4, body, 0)


@functools.partial(
    pl.kernel,
    out_type=jax.ShapeDtypeStruct((COLS, 4, NW, 8, RB), jnp.float32),
    mesh=_mesh,
    compiler_params=pltpu.CompilerParams(use_tc_tiling_on_sc=False,
                                         needs_layout_passes=False),
    scratch_types=[
        pltpu.VMEM((CB, 8, RB), jnp.int32),
        *[pltpu.VMEM((RB, DIM), jnp.float32) for _ in range(NBUF)],
        *[pltpu.VMEM((4, 8, RB + 1), jnp.float32) for _ in range(NBUF)],
        *[pltpu.SemaphoreType.DMA for _ in range(NBUF)],
        *[pltpu.SemaphoreType.DMA for _ in range(NBUF)],
    ],
)
def _emb_lookup(x_hbm, table_hbm, out_hbm, idx_v, *bufs_and_sems):
    in_bufs = bufs_and_sems[:NBUF]
    out_bufs = bufs_and_sems[NBUF:2 * NBUF]
    gsems = bufs_and_sems[2 * NBUF:3 * NBUF]
    ssems = bufs_and_sems[3 * NBUF:4 * NBUF]

    wid = lax.axis_index("s") * 2 + lax.axis_index("c")

    pltpu.sync_copy(x_hbm.at[:, wid], idx_v)

    def _remap_idx(r, carry):
        cb = r // 8
        a = r % 8
        for h in range(RB // 16):
            v = idx_v[cb, a, pl.ds(16 * h, 16)]
            q = ((v >= VQ).astype(jnp.int32)
                 + (v >= 2 * VQ).astype(jnp.int32)
                 + (v >= 3 * VQ).astype(jnp.int32))
            idx_v[cb, a, pl.ds(16 * h, 16)] = (v - q * VQ) * 4 + q
        return carry

    lax.fori_loop(0, CB * 8, _remap_idx, 0)

    def _idx_slice(c):
        return idx_v.at[c // 8, c % 8]

    def _out_slice(c):
        return out_hbm.at[c, :, wid]

    for b in range(NBUF):
        pltpu.async_copy(table_hbm.at[_idx_slice(b)],
                         in_bufs[b], gsems[b])

    def group(g, carry):
        for b in range(NBUF):
            c = g * NBUF + b
            pltpu.make_async_copy(
                table_hbm.at[_idx_slice(c)],
                in_bufs[b], gsems[b]).wait()
            @pl.when(g > 0)
            def _():
                pltpu.make_async_copy(out_bufs[b].at[:, :, pl.ds(0, RB)],
                                      _out_slice(c - NBUF), ssems[b]).wait()

            _transpose_scale(in_bufs[b], out_bufs[b])

            @pl.when(g < NGRP - 1)
            def _():
                pltpu.async_copy(
                    table_hbm.at[_idx_slice(c + NBUF)],
                    in_bufs[b], gsems[b])

            pltpu.async_copy(out_bufs[b].at[:, :, pl.ds(0, RB)],
                             _out_slice(c), ssems[b])
        return carry

    lax.fori_loop(0, NGRP, group, 0)

    for b in range(NBUF):
        c = (NGRP - 1) * NBUF + b
        pltpu.make_async_copy(out_bufs[b].at[:, :, pl.ds(0, RB)],
                              _out_slice(c), ssems[b]).wait()


TW = 1024
TGRID = 245
VQ = TW * TGRID


def _tc_transpose_body(t0, t1, t2, t3, out_ref):
    eye = jnp.eye(DIM, dtype=jnp.float32) * SCALE
    dn = (((0,), (0,)), ((), ()))

    def tr(t):
        return lax.dot_general(t[...], eye, dn,
                               preferred_element_type=jnp.float32)

    out_ref[...] = jnp.concatenate([tr(t0), tr(t1), tr(t2), tr(t3)], axis=1)


_tc_transpose = pl.pallas_call(
    _tc_transpose_body,
    grid=(TGRID,),
    in_specs=[
        pl.BlockSpec((DIM, TW), functools.partial(
            lambda j, i: (0, jnp.minimum(j * TGRID + i,
                                         (VOCAB - 1) // TW)), j))
        for j in range(4)
    ],
    out_specs=pl.BlockSpec((TW, 4 * DIM), lambda i: (i, 0)),
    out_shape=jax.ShapeDtypeStruct((VQ, 4 * DIM), jnp.float32),
)


def kernel(x, table):
    xp = x.astype(jnp.int32).T.reshape(CB, 8, NW, RB).transpose(0, 2, 1, 3)
    tt = table.T
    t2 = _tc_transpose(tt, tt, tt, tt)
    out5 = _emb_lookup(xp, t2.reshape(4 * VQ, DIM))
    return out5.transpose(2, 4, 0, 1, 3).reshape(ROWS, COLS, DIM)


# speedup vs baseline: 1.6559x; 1.0805x over previous
import functools
import math

import jax
import jax.numpy as jnp
from jax import lax
from jax.experimental import pallas as pl
from jax.experimental.pallas import tpu as pltpu
from jax.experimental.pallas import tpu_sc as plsc

VOCAB = 1000000
DIM = 32
ROWS = 4096
COLS = 200
SCALE = math.sqrt(float(DIM))

NW = 32
RB = ROWS // NW
CB = COLS // 8
NBUF = 5
NGRP = COLS // NBUF

_mesh = plsc.VectorSubcoreMesh(core_axis_name="c", subcore_axis_name="s")


def _transpose_scale(src, dst):
    lanes = lax.iota(jnp.int32, 16)

    def body(g, carry):
        for u in range(4):
            r = g * 4 + u
            rvec = jnp.full((16,), r, jnp.int32)
            for h in range(2):
                d = h * 16 + lanes
                ti_idx = lax.shift_right_logical(d, 3)
                f_idx = lax.bitwise_and(d, 7)
                v = src[r, pl.ds(16 * h, 16)]
                plsc.store_scatter(dst, [ti_idx, f_idx, rvec], v)
        return carry

    lax.fori_loop(0, RB // 4, body, 0)


@functools.partial(
    pl.kernel,
    out_type=jax.ShapeDtypeStruct((COLS, 4, NW, 8, RB), jnp.float32),
    mesh=_mesh,
    compiler_params=pltpu.CompilerParams(use_tc_tiling_on_sc=False,
                                         needs_layout_passes=False),
    scratch_types=[
        pltpu.VMEM((CB, 8, RB), jnp.int32),
        *[pltpu.VMEM((RB, DIM), jnp.float32) for _ in range(NBUF)],
        *[pltpu.VMEM((4, 8, RB + 1), jnp.float32) for _ in range(NBUF)],
        *[pltpu.SemaphoreType.DMA for _ in range(NBUF)],
        *[pltpu.SemaphoreType.DMA for _ in range(NBUF)],
    ],
)
def _emb_lookup(x_hbm, table_hbm, out_hbm, idx_v, *bufs_and_sems):
    in_bufs = bufs_and_sems[:NBUF]
    out_bufs = bufs_and_sems[NBUF:2 * NBUF]
    gsems = bufs_and_sems[2 * NBUF:3 * NBUF]
    ssems = bufs_and_sems[3 * NBUF:4 * NBUF]

    wid = lax.axis_index("s") * 2 + lax.axis_index("c")

    pltpu.sync_copy(x_hbm.at[:, wid], idx_v)

    def _remap_idx(r, carry):
        cb = r // 8
        a = r % 8
        for h in range(RB // 16):
            v = idx_v[cb, a, pl.ds(16 * h, 16)]
            q = ((v >= VQ).astype(jnp.int32)
                 + (v >= 2 * VQ).astype(jnp.int32)
                 + (v >= 3 * VQ).astype(jnp.int32))
            idx_v[cb, a, pl.ds(16 * h, 16)] = (v - q * VQ) * 4 + q
        return carry

    lax.fori_loop(0, CB * 8, _remap_idx, 0)

    def _idx_slice(c):
        return idx_v.at[c // 8, c % 8]

    def _out_slice(c):
        return out_hbm.at[c, :, wid]

    for b in range(NBUF):
        pltpu.async_copy(table_hbm.at[_idx_slice(b)],
                         in_bufs[b], gsems[b])

    def group(g, carry):
        for b in range(NBUF):
            c = g * NBUF + b
            pltpu.make_async_copy(
                table_hbm.at[_idx_slice(c)],
                in_bufs[b], gsems[b]).wait()
            @pl.when(g > 0)
            def _():
                pltpu.make_async_copy(out_bufs[b].at[:, :, pl.ds(0, RB)],
                                      _out_slice(c - NBUF), ssems[b]).wait()

            _transpose_scale(in_bufs[b], out_bufs[b])

            @pl.when(g < NGRP - 1)
            def _():
                pltpu.async_copy(
                    table_hbm.at[_idx_slice(c + NBUF)],
                    in_bufs[b], gsems[b])

            pltpu.async_copy(out_bufs[b].at[:, :, pl.ds(0, RB)],
                             _out_slice(c), ssems[b])
        return carry

    lax.fori_loop(0, NGRP, group, 0)

    for b in range(NBUF):
        c = (NGRP - 1) * NBUF + b
        pltpu.make_async_copy(out_bufs[b].at[:, :, pl.ds(0, RB)],
                              _out_slice(c), ssems[b]).wait()


TW = 2048
TGRID = 123
VQ = TW * TGRID


def _tc_transpose_body(t0, t1, t2, t3, out_ref):
    eye = jnp.eye(DIM, dtype=jnp.float32) * SCALE
    dn = (((0,), (0,)), ((), ()))

    def tr(t):
        return lax.dot_general(t[...], eye, dn,
                               preferred_element_type=jnp.float32)

    out_ref[...] = jnp.concatenate([tr(t0), tr(t1), tr(t2), tr(t3)], axis=1)


_tc_transpose = pl.pallas_call(
    _tc_transpose_body,
    grid=(TGRID,),
    in_specs=[
        pl.BlockSpec((DIM, TW), functools.partial(
            lambda j, i: (0, jnp.minimum(j * TGRID + i,
                                         (VOCAB - 1) // TW)), j))
        for j in range(4)
    ],
    out_specs=pl.BlockSpec((TW, 4 * DIM), lambda i: (i, 0)),
    out_shape=jax.ShapeDtypeStruct((VQ, 4 * DIM), jnp.float32),
)


def kernel(x, table):
    xp = x.astype(jnp.int32).T.reshape(CB, 8, NW, RB).transpose(0, 2, 1, 3)
    tt = table.T
    t2 = _tc_transpose(tt, tt, tt, tt)
    out5 = _emb_lookup(xp, t2.reshape(4 * VQ, DIM))
    return out5.transpose(2, 4, 0, 1, 3).reshape(ROWS, COLS, DIM)


# speedup vs baseline: 1.6831x; 1.0165x over previous
import functools
import math

import jax
import jax.numpy as jnp
from jax import lax
from jax.experimental import pallas as pl
from jax.experimental.pallas import tpu as pltpu
from jax.experimental.pallas import tpu_sc as plsc

VOCAB = 1000000
DIM = 32
ROWS = 4096
COLS = 200
SCALE = math.sqrt(float(DIM))

NW = 32
RB = ROWS // NW
CB = COLS // 8
NBUF = 8
NGRP = COLS // NBUF

_mesh = plsc.VectorSubcoreMesh(core_axis_name="c", subcore_axis_name="s")


def _transpose_scale(src, dst):
    lanes = lax.iota(jnp.int32, 16)

    def body(g, carry):
        for u in range(4):
            r = g * 4 + u
            rvec = jnp.full((16,), r, jnp.int32)
            for h in range(2):
                d = h * 16 + lanes
                ti_idx = lax.shift_right_logical(d, 3)
                f_idx = lax.bitwise_and(d, 7)
                v = src[r, pl.ds(16 * h, 16)]
                plsc.store_scatter(dst, [ti_idx, f_idx, rvec], v)
        return carry

    lax.fori_loop(0, RB // 4, body, 0)


@functools.partial(
    pl.kernel,
    out_type=jax.ShapeDtypeStruct((COLS, 4, NW, 8, RB), jnp.float32),
    mesh=_mesh,
    compiler_params=pltpu.CompilerParams(use_tc_tiling_on_sc=False,
                                         needs_layout_passes=False),
    scratch_types=[
        pltpu.VMEM((CB, 8, RB), jnp.int32),
        *[pltpu.VMEM((RB, DIM), jnp.float32) for _ in range(NBUF)],
        *[pltpu.VMEM((4, 8, RB + 1), jnp.float32) for _ in range(NBUF)],
        *[pltpu.SemaphoreType.DMA for _ in range(NBUF)],
        *[pltpu.SemaphoreType.DMA for _ in range(NBUF)],
    ],
)
def _emb_lookup(x_hbm, table_hbm, out_hbm, idx_v, *bufs_and_sems):
    in_bufs = bufs_and_sems[:NBUF]
    out_bufs = bufs_and_sems[NBUF:2 * NBUF]
    gsems = bufs_and_sems[2 * NBUF:3 * NBUF]
    ssems = bufs_and_sems[3 * NBUF:4 * NBUF]

    wid = lax.axis_index("s") * 2 + lax.axis_index("c")

    pltpu.sync_copy(x_hbm.at[:, wid], idx_v)

    def _remap_idx(r, carry):
        cb = r // 8
        a = r % 8
        for h in range(RB // 16):
            v = idx_v[cb, a, pl.ds(16 * h, 16)]
            q = ((v >= VQ).astype(jnp.int32)
                 + (v >= 2 * VQ).astype(jnp.int32)
                 + (v >= 3 * VQ).astype(jnp.int32))
            idx_v[cb, a, pl.ds(16 * h, 16)] = (v - q * VQ) * 4 + q
        return carry

    lax.fori_loop(0, CB * 8, _remap_idx, 0)

    def _idx_slice(c):
        return idx_v.at[c // 8, c % 8]

    def _out_slice(c):
        return out_hbm.at[c, :, wid]

    for b in range(NBUF):
        pltpu.async_copy(table_hbm.at[_idx_slice(b)],
                         in_bufs[b], gsems[b])

    def group(g, carry):
        for b in range(NBUF):
            c = g * NBUF + b
            pltpu.make_async_copy(
                table_hbm.at[_idx_slice(c)],
                in_bufs[b], gsems[b]).wait()
            @pl.when(g > 0)
            def _():
                pltpu.make_async_copy(out_bufs[b].at[:, :, pl.ds(0, RB)],
                                      _out_slice(c - NBUF), ssems[b]).wait()

            _transpose_scale(in_bufs[b], out_bufs[b])

            @pl.when(g < NGRP - 1)
            def _():
                pltpu.async_copy(
                    table_hbm.at[_idx_slice(c + NBUF)],
                    in_bufs[b], gsems[b])

            pltpu.async_copy(out_bufs[b].at[:, :, pl.ds(0, RB)],
                             _out_slice(c), ssems[b])
        return carry

    lax.fori_loop(0, NGRP, group, 0)

    for b in range(NBUF):
        c = (NGRP - 1) * NBUF + b
        pltpu.make_async_copy(out_bufs[b].at[:, :, pl.ds(0, RB)],
                              _out_slice(c), ssems[b]).wait()


TW = 4096
TGRID = 62
VQ = TW * TGRID


def _tc_transpose_body(t0, t1, t2, t3, out_ref):
    eye = jnp.eye(DIM, dtype=jnp.float32) * SCALE
    dn = (((0,), (0,)), ((), ()))

    def tr(t):
        return lax.dot_general(t[...], eye, dn,
                               preferred_element_type=jnp.float32)

    out_ref[...] = jnp.concatenate([tr(t0), tr(t1), tr(t2), tr(t3)], axis=1)


_tc_transpose = pl.pallas_call(
    _tc_transpose_body,
    grid=(TGRID,),
    in_specs=[
        pl.BlockSpec((DIM, TW), functools.partial(
            lambda j, i: (0, jnp.minimum(j * TGRID + i,
                                         (VOCAB - 1) // TW)), j))
        for j in range(4)
    ],
    out_specs=pl.BlockSpec((TW, 4 * DIM), lambda i: (i, 0)),
    out_shape=jax.ShapeDtypeStruct((VQ, 4 * DIM), jnp.float32),
)


def kernel(x, table):
    xp = x.astype(jnp.int32).T.reshape(CB, 8, NW, RB).transpose(0, 2, 1, 3)
    tt = table.T
    t2 = _tc_transpose(tt, tt, tt, tt)
    out5 = _emb_lookup(xp, t2.reshape(4 * VQ, DIM))
    return out5.transpose(2, 4, 0, 1, 3).reshape(ROWS, COLS, DIM)
